# Initial kernel scaffold; baseline (speedup 1.0000x reference)
import jax
import jax.numpy as jnp
from jax.experimental import pallas as pl


def kernel(input, adj, W_conv, b_conv, bias):
    raise NotImplementedError("write your pallas kernel here")


# speedup vs baseline: 15.6359x; 15.6359x over previous
import functools

import jax
import jax.numpy as jnp
from jax import lax
from jax.experimental import pallas as pl
from jax.experimental.pallas import tpu as pltpu
from jax.experimental.pallas import tpu_sc as plsc

_NC = 2
_NS = 16
_B = 128
_IBLK = 16


def _sc_degree(cidx, npad, c0, c1):
    mesh = plsc.VectorSubcoreMesh(core_axis_name="c", subcore_axis_name="s")
    lanes = 16

    @functools.partial(
        pl.kernel,
        out_type=jax.ShapeDtypeStruct((_NC, _NS, npad), jnp.float32),
        mesh=mesh,
        compiler_params=pltpu.CompilerParams(needs_layout_passes=False),
        scratch_types=[
            pltpu.VMEM((c0, _B), jnp.int32),
            pltpu.VMEM((npad,), jnp.float32),
        ],
    )
    def deg_kernel(cidx_hbm, out_hbm, cidx_vm, degv):
        c = lax.axis_index("c")
        s = lax.axis_index("s")
        base = jnp.where(c == 0, s * c0, _NS * c0 + s * c1)
        mycd = jnp.where(c == 0, c0, c1)
        pltpu.sync_copy(cidx_hbm.at[pl.ds(base, c0)], cidx_vm)
        zeros16 = jnp.zeros((lanes,), jnp.float32)
        ones16 = jnp.ones((lanes,), jnp.float32)

        def zbody(i, carry):
            degv[pl.ds(i * lanes, lanes)] = zeros16
            return carry

        lax.fori_loop(0, npad // lanes, zbody, 0)

        def body(j, carry):
            def inner(kk, carry2):
                idx = cidx_vm[j, pl.ds(kk * lanes, lanes)]
                plsc.addupdate_scatter(degv, [idx], ones16)
                return carry2

            lax.fori_loop(0, _B // lanes, inner, 0)
            return carry

        lax.fori_loop(0, mycd, body, 0)
        pltpu.sync_copy(degv, out_hbm.at[c, s])

    return deg_kernel(cidx)


def _sc_aggregate(y, eidx, zeros_v, npad, ch0, ch1, d):
    stripe = npad // _NS
    mesh = plsc.VectorSubcoreMesh(core_axis_name="c", subcore_axis_name="s")

    @functools.partial(
        pl.kernel,
        out_type=jax.ShapeDtypeStruct((_NC, npad, d), jnp.float32),
        mesh=mesh,
        scratch_types=[
            pltpu.VMEM((_IBLK, 2, _B), jnp.int32),
            pltpu.VMEM((_IBLK, 2, _B), jnp.int32),
            pltpu.VMEM((2, _B, d), jnp.float32),
            pltpu.VMEM_SHARED((npad, d), jnp.float32),
            [pltpu.SemaphoreType.DMA] * 2,
            [pltpu.SemaphoreType.DMA] * 2,
        ],
    )
    def agg_kernel(y_hbm, eidx_hbm, z_hbm, out_hbm,
                   ibuf0, ibuf1, gbuf, acc, semi, semg):
        c = lax.axis_index("c")
        s = lax.axis_index("s")
        base = jnp.where(c == 0, s * ch0, _NS * ch0 + s * ch1)
        nq = jnp.where(c == 0, ch0 // (2 * _IBLK), ch1 // (2 * _IBLK))
        pltpu.async_copy(eidx_hbm.at[pl.ds(base, _IBLK)], ibuf0, semi[0])
        pltpu.sync_copy(z_hbm, acc.at[pl.ds(s * stripe, stripe)])
        plsc.subcore_barrier()

        def run_block(ibuf):
            def inner(m, carry):
                j0 = 2 * m
                d0 = pltpu.async_copy(y_hbm.at[ibuf.at[j0, 0]],
                                      gbuf.at[0], semg[0])
                d1 = pltpu.async_copy(y_hbm.at[ibuf.at[j0 + 1, 0]],
                                      gbuf.at[1], semg[1])
                d0.wait()
                pltpu.sync_copy(gbuf.at[0], acc.at[ibuf.at[j0, 1]], add=True)
                d1.wait()
                pltpu.sync_copy(gbuf.at[1], acc.at[ibuf.at[j0 + 1, 1]], add=True)
                return carry

            lax.fori_loop(0, _IBLK // 2, inner, 0)

        def outer(q, carry):
            bb = base + q * (2 * _IBLK)
            pltpu.async_copy(eidx_hbm.at[pl.ds(bb + _IBLK, _IBLK)], ibuf1, semi[1])
            pltpu.make_async_copy(eidx_hbm.at[pl.ds(bb, _IBLK)], ibuf0, semi[0]).wait()
            run_block(ibuf0)

            @pl.when(q + 1 < nq)
            def _():
                pltpu.async_copy(eidx_hbm.at[pl.ds(bb + 2 * _IBLK, _IBLK)],
                                 ibuf0, semi[0])

            pltpu.make_async_copy(eidx_hbm.at[pl.ds(bb + _IBLK, _IBLK)],
                                  ibuf1, semi[1]).wait()
            run_block(ibuf1)
            return carry

        lax.fori_loop(0, nq, outer, 0)
        plsc.subcore_barrier()
        pltpu.sync_copy(acc.at[pl.ds(s * stripe, stripe)],
                        out_hbm.at[c, pl.ds(s * stripe, stripe)])

    return agg_kernel(y, eidx, zeros_v)


def _tc_transform(x, w, degparts, b_conv, bias, blk):
    n, d_in = x.shape
    d = w.shape[1]

    def body(x_ref, w_ref, dp_ref, b1_ref, b2_ref, y_ref, self_ref):
        xw = jnp.dot(x_ref[...], w_ref[...], preferred_element_type=jnp.float32)
        deg = 1.0 + jnp.sum(dp_ref[...], axis=1)
        dinv = lax.rsqrt(deg)[:, None]
        y_ref[...] = dinv * xw
        self_ref[...] = (dinv * dinv) * xw + b1_ref[...] + b2_ref[...]

    return pl.pallas_call(
        body,
        grid=(n // blk,),
        in_specs=[
            pl.BlockSpec((blk, d_in), lambda i: (i, 0)),
            pl.BlockSpec((d_in, d), lambda i: (0, 0)),
            pl.BlockSpec((blk, _NC * _NS), lambda i: (i, 0)),
            pl.BlockSpec((1, d), lambda i: (0, 0)),
            pl.BlockSpec((1, d), lambda i: (0, 0)),
        ],
        out_specs=[pl.BlockSpec((blk, d), lambda i: (i, 0)),
                   pl.BlockSpec((blk, d), lambda i: (i, 0))],
        out_shape=[jax.ShapeDtypeStruct((n, d), jnp.float32),
                   jax.ShapeDtypeStruct((n, d), jnp.float32)],
    )(x, w, degparts, b_conv.reshape(1, d), bias.reshape(1, d))


def _tc_combine(accparts, degparts, selfterm, blk):
    n, d = selfterm.shape

    def body(a_ref, dp_ref, s_ref, o_ref):
        a = a_ref[...]
        deg = 1.0 + jnp.sum(dp_ref[...], axis=1)
        o_ref[...] = lax.rsqrt(deg)[:, None] * (a[0] + a[1]) + s_ref[...]

    return pl.pallas_call(
        body,
        grid=(n // blk,),
        in_specs=[
            pl.BlockSpec((_NC, blk, d), lambda i: (0, i, 0)),
            pl.BlockSpec((blk, _NC * _NS), lambda i: (i, 0)),
            pl.BlockSpec((blk, d), lambda i: (i, 0)),
        ],
        out_specs=pl.BlockSpec((blk, d), lambda i: (i, 0)),
        out_shape=jax.ShapeDtypeStruct((n, d), jnp.float32),
    )(accparts, degparts, selfterm)


def kernel(input, adj, W_conv, b_conv, bias):
    x = input
    n, d_in = x.shape
    d = W_conv.shape[1]
    e = adj.shape[1]

    nb = -(-e // _B)
    unit = 2 * _IBLK
    ch0 = -(-(nb * 5) // (7 * _NS))
    ch0 = -(-ch0 // unit) * unit
    ch1 = -(-max(0, nb - _NS * ch0) // _NS)
    ch1 = max(unit, -(-ch1 // unit) * unit)
    nbp = _NS * (ch0 + ch1)
    epad = nbp * _B
    c0d = -(-(-(-(nb * 2) // (3 * _NS))) // 8) * 8
    c1d = -(-max(0, nb - _NS * c0d) // _NS)
    c1d = max(8, -(-c1d // 8) * 8)
    nbd = _NS * c0d + (_NS - 1) * c1d + c0d
    epad_deg = nbd * _B

    stripe = -(-(n + 8) // _NS)
    stripe = -(-stripe // 8) * 8
    npad = stripe * _NS
    blk = 1000 if n % 1000 == 0 else n

    row = adj[0]
    col = adj[1]
    rowp = jnp.concatenate([row, jnp.zeros((epad - e,), jnp.int32)])
    colp = jnp.concatenate([col, jnp.full((epad - e,), n, jnp.int32)])
    eidx = jnp.stack([rowp.reshape(nbp, _B), colp.reshape(nbp, _B)], axis=1)
    cidx_deg = jnp.concatenate(
        [col, jnp.full((epad_deg - e,), n, jnp.int32)]
    ).reshape(nbd, _B)

    zacc = jnp.zeros((stripe, d), jnp.float32)

    degparts = _sc_degree(cidx_deg, npad, c0d, c1d).reshape(_NC * _NS, npad).T
    y, selfterm = _tc_transform(x, W_conv, degparts, b_conv, bias, blk)
    accparts = _sc_aggregate(y, eidx, zacc, npad, ch0, ch1, d)
    return _tc_combine(accparts, degparts, selfterm, blk)
